# baseline (speedup 1.0000x reference)
import functools

import numpy as np

import jax
import jax.numpy as jnp
from jax import lax
from jax.experimental import pallas as pl
from jax.experimental.pallas import tpu as pltpu
from jax.experimental.pallas import tpu_sc as plsc

C = 256
S = 16384
T = 1024
D = 31
B = 1024

SROWS = 128

NW = 32
RPW = B // NW
L = 16
UNROLL = 8


def _np_sampled_counts() -> np.ndarray:
    def rotl(x, r):
        return ((x << np.uint32(r)) | (x >> np.uint32(32 - r))).astype(np.uint32)

    ks = [np.uint32(0), np.uint32(42), np.uint32(0x1BD11BDA) ^ np.uint32(42)]
    x0 = np.zeros(S, np.uint32) + ks[0]
    x1 = (np.arange(S, dtype=np.uint32) + ks[1]).astype(np.uint32)
    rotations = [(13, 15, 26, 6), (17, 29, 16, 24)]
    for i in range(5):
        for r in rotations[i % 2]:
            x0 = (x0 + x1).astype(np.uint32)
            x1 = rotl(x1, r) ^ x0
        x0 = (x0 + ks[(i + 1) % 3]).astype(np.uint32)
        x1 = (x1 + ks[(i + 2) % 3] + np.uint32(i + 1)).astype(np.uint32)
    bits = x0 ^ x1
    u = (((bits >> np.uint32(9)) | np.uint32(0x3F800000)).view(np.float32)
         - np.float32(1.0))
    ids = np.clip((np.exp(u * np.log(np.float32(C) + 1.0)) - 1.0)
                  .astype(np.int32), 0, C - 1)
    return np.bincount(ids, minlength=C).astype(np.float32).reshape(C, 1)


_CNT = _np_sampled_counts()


def _tc_body(x_ref, w_ref, b_ref, cnt_ref, g_ref):
    cnt = cnt_ref[...]
    x = x_ref[...]
    w = w_ref[...]
    logits = lax.dot_general(x, w, (((1,), (1,)), ((), ())),
                             preferred_element_type=jnp.float32)
    ci = lax.broadcasted_iota(jnp.int32, (1, C), 1).astype(jnp.float32)
    q = (jnp.log(ci + 2.0) - jnp.log(ci + 1.0)) / jnp.log(float(C) + 1.0)
    adj = logits + b_ref[...] - jnp.log(float(S) * q)
    sp = jnp.maximum(adj, 0.0) + jnp.log1p(jnp.exp(-jnp.abs(adj)))
    part = lax.dot_general(sp, cnt, (((1,), (0,)), ((), ())),
                           preferred_element_type=jnp.float32)
    g_ref[...] = sp - adj * (1.0 / T) + part * (1.0 / T)


def _tc_tables(x, w, b2, cnt):
    return pl.pallas_call(
        _tc_body,
        out_shape=jax.ShapeDtypeStruct((B, C), jnp.float32),
    )(x, w, b2, cnt)


def _sc_body(g_hbm, labels_hbm, out_hbm, lab_v, g_v, out_v, sem):
    wid = lax.axis_index("s") * 2 + lax.axis_index("c")
    base = wid * RPW
    cp_lab = pltpu.async_copy(labels_hbm.at[pl.ds(base, RPW), :], lab_v, sem)
    cp_g = pltpu.async_copy(g_hbm.at[pl.ds(base, RPW), :], g_v, sem)
    cp_lab.wait()
    cp_g.wait()

    lanes = lax.iota(jnp.int32, L)

    for grp in range(RPW // L):
        def row_body(r16, outvec, grp=grp):
            r = grp * L + r16
            rsplat = jnp.full((L,), 0, jnp.int32) + r

            acc0 = jnp.zeros((L,), jnp.float32)
            acc1 = jnp.zeros((L,), jnp.float32)
            for k in range(T // (2 * L)):
                i0 = lab_v[r, pl.ds(2 * k * L, L)]
                i1 = lab_v[r, pl.ds((2 * k + 1) * L, L)]
                acc0 = acc0 + plsc.load_gather(g_v, [rsplat, i0])
                acc1 = acc1 + plsc.load_gather(g_v, [rsplat, i1])
            return outvec + jnp.where(lanes == r16, jnp.sum(acc0 + acc1), 0.0)

        outvec = lax.fori_loop(0, L, row_body, jnp.zeros((L,), jnp.float32))
        out_v[pl.ds(grp * L, L)] = outvec

    pltpu.sync_copy(out_v, out_hbm.at[pl.ds(base, RPW)])


_sc_true_sum = functools.partial(
    pl.kernel,
    out_type=jax.ShapeDtypeStruct((B,), jnp.float32),
    mesh=plsc.VectorSubcoreMesh(core_axis_name="c", subcore_axis_name="s"),
    compiler_params=pltpu.CompilerParams(use_tc_tiling_on_sc=False,
                                         needs_layout_passes=False,
                                         skip_device_barrier=True),
    scratch_types=[
        pltpu.VMEM((RPW, T), jnp.int32),
        pltpu.VMEM((RPW, C), jnp.float32),
        pltpu.VMEM((RPW,), jnp.float32),
        pltpu.SemaphoreType.DMA,
    ],
)(_sc_body)


def kernel(inputs, labels, w, b):
    labels = labels.astype(jnp.int32)
    b2 = b.reshape(1, C)
    g = _tc_tables(inputs, w, b2, jnp.asarray(_CNT))
    return _sc_true_sum(g, labels)
